# baseline (speedup 1.0000x reference)
import jax
import jax.numpy as jnp
from jax.experimental import pallas as pl
from jax.experimental.pallas import tpu as pltpu

N = 8192
D = 64
BCA = 512
BRB = 512
NBA = N // BCA
NBB = N // BRB
NEG_SLOPE = 0.01


def _pass_a(adj_ref, embs_t_ref, w_ref, y_ref, acc_ref, sr_ref):
    j = pl.program_id(0)

    @pl.when(j == 0)
    def _init():
        xt = embs_t_ref[...]
        rn = jnp.sqrt(jnp.sum(xt * xt, axis=0, keepdims=True))
        sr_ref[...] = jnp.sum(xt, axis=0, keepdims=True) / jnp.maximum(rn, 1e-12)
        acc_ref[...] = jnp.zeros_like(acc_ref)

    a = adj_ref[...]
    csq_acc = jnp.zeros((128, BCA), dtype=jnp.float32)
    for r in range(0, N, 128):
        c = a[r:r + 128, :]
        csq_acc = csq_acc + c * c
    csq = jnp.sum(csq_acc, axis=0, keepdims=True)
    s_blk = sr_ref[:, pl.ds(j * BCA, BCA)]
    t_row = s_blk / jnp.maximum(jnp.sqrt(csq), 1e-12)

    acc = acc_ref[...]
    for k in range(BCA // 128):
        acc = acc + a[:, k * 128:(k + 1) * 128] * t_row[:, k * 128:(k + 1) * 128]
    acc_ref[...] = acc

    @pl.when(j == NBA - 1)
    def _finalize():
        nm = jnp.sum(acc_ref[...], axis=1, keepdims=True)
        total = jnp.sum(nm)
        h = jax.lax.dot_general(
            embs_t_ref[...], w_ref[...], (((0,), (0,)), ((), ())),
            preferred_element_type=jnp.float32,
        )
        y_ref[...] = (nm * (1.0 / total)) * h


def _pass_b(adj_ref, y_ref, out_ref):
    o = jnp.dot(adj_ref[...], y_ref[...], preferred_element_type=jnp.float32)
    out_ref[...] = jnp.where(o >= 0, o, NEG_SLOPE * o)


@jax.jit
def _run(adj_matrix, node_embs, W):
    embs_t = node_embs.T

    y = pl.pallas_call(
        _pass_a,
        grid=(NBA,),
        in_specs=[
            pl.BlockSpec((N, BCA), lambda j: (0, j)),
            pl.BlockSpec((D, N), lambda j: (0, 0)),
            pl.BlockSpec((D, D), lambda j: (0, 0)),
        ],
        out_specs=pl.BlockSpec((N, D), lambda j: (0, 0)),
        out_shape=jax.ShapeDtypeStruct((N, D), jnp.float32),
        scratch_shapes=[
            pltpu.VMEM((N, 128), jnp.float32),
            pltpu.VMEM((1, N), jnp.float32),
        ],
        compiler_params=pltpu.CompilerParams(
            dimension_semantics=("arbitrary",),
        ),
    )(adj_matrix, embs_t, W)

    out = pl.pallas_call(
        _pass_b,
        grid=(NBB,),
        in_specs=[
            pl.BlockSpec((BRB, N), lambda i: (i, 0)),
            pl.BlockSpec((N, D), lambda i: (0, 0)),
        ],
        out_specs=pl.BlockSpec((BRB, D), lambda i: (i, 0)),
        out_shape=jax.ShapeDtypeStruct((N, D), jnp.float32),
        compiler_params=pltpu.CompilerParams(
            dimension_semantics=("arbitrary",),
        ),
    )(adj_matrix, y)
    return out


def kernel(adj_matrix, node_embs, W):
    return _run(adj_matrix, node_embs, W)


# speedup vs baseline: 1.0273x; 1.0273x over previous
import jax
import jax.numpy as jnp
from jax.experimental import pallas as pl
from jax.experimental.pallas import tpu as pltpu

N = 8192
D = 64
BCA = 256
BRB = 256
NBA = N // BCA
NBB = N // BRB
NEG_SLOPE = 0.01


def _fused(adj_col_ref, adj_row_ref, embs_t_ref, w_ref, out_ref,
           acc_ref, sr_ref, y_ref):
    j = pl.program_id(0)

    @pl.when(j == 0)
    def _init():
        xt = embs_t_ref[...]
        rn = jnp.sqrt(jnp.sum(xt * xt, axis=0, keepdims=True))
        sr_ref[...] = jnp.sum(xt, axis=0, keepdims=True) / jnp.maximum(rn, 1e-12)
        acc_ref[...] = jnp.zeros_like(acc_ref)

    @pl.when(j < NBA)
    def _phase_a():
        a = adj_col_ref[...]
        csq_acc = jnp.zeros((128, BCA), dtype=jnp.float32)
        for r in range(0, N, 128):
            c = a[r:r + 128, :]
            csq_acc = csq_acc + c * c
        csq = jnp.sum(csq_acc, axis=0, keepdims=True)
        s_blk = sr_ref[:, pl.ds(j * BCA, BCA)]
        t_row = s_blk / jnp.maximum(jnp.sqrt(csq), 1e-12)

        acc = acc_ref[...]
        for k in range(BCA // 128):
            acc = acc + a[:, k * 128:(k + 1) * 128] * t_row[:, k * 128:(k + 1) * 128]
        acc_ref[...] = acc

        @pl.when(j == NBA - 1)
        def _finalize():
            nm = jnp.sum(acc_ref[...], axis=1, keepdims=True)
            total = jnp.sum(nm)
            h = jax.lax.dot_general(
                embs_t_ref[...], w_ref[...], (((0,), (0,)), ((), ())),
                preferred_element_type=jnp.float32,
            )
            y_ref[...] = (nm * (1.0 / total)) * h

    @pl.when(j >= NBA)
    def _phase_b():
        o = jnp.dot(adj_row_ref[...], y_ref[...],
                    preferred_element_type=jnp.float32)
        out_ref[...] = jnp.where(o >= 0, o, NEG_SLOPE * o)


@jax.jit
def _run(adj_matrix, node_embs, W):
    embs_t = node_embs.T

    def _cap(v, hi):
        return jnp.minimum(v, hi)

    return pl.pallas_call(
        _fused,
        grid=(NBA + NBB,),
        in_specs=[
            pl.BlockSpec((N, BCA), lambda j: (0, _cap(j, NBA - 1))),
            pl.BlockSpec((BRB, N),
                         lambda j: (_cap(jnp.maximum(j - NBA, 0), NBB - 1), 0)),
            pl.BlockSpec((D, N), lambda j: (0, 0)),
            pl.BlockSpec((D, D), lambda j: (0, 0)),
        ],
        out_specs=pl.BlockSpec(
            (BRB, D), lambda j: (_cap(jnp.maximum(j - NBA, 0), NBB - 1), 0)),
        out_shape=jax.ShapeDtypeStruct((N, D), jnp.float32),
        scratch_shapes=[
            pltpu.VMEM((N, 128), jnp.float32),
            pltpu.VMEM((1, N), jnp.float32),
            pltpu.VMEM((N, D), jnp.float32),
        ],
        compiler_params=pltpu.CompilerParams(
            dimension_semantics=("arbitrary",),
        ),
    )(adj_matrix, adj_matrix, embs_t, W)


def kernel(adj_matrix, node_embs, W):
    return _run(adj_matrix, node_embs, W)


# speedup vs baseline: 1.0290x; 1.0017x over previous
import jax
import jax.numpy as jnp
from jax.experimental import pallas as pl
from jax.experimental.pallas import tpu as pltpu

N = 8192
D = 64
BCA = 256
BRB = 256
NBA = N // BCA
NBB = N // BRB
NEG_SLOPE = 0.01


def _fused(adj_col_ref, adj_row_ref, embs_t_ref, w_ref, out_ref,
           acc_ref, sr_ref):
    j = pl.program_id(0)

    @pl.when(j == 0)
    def _init():
        xt = embs_t_ref[...]
        rn = jnp.sqrt(jnp.sum(xt * xt, axis=0, keepdims=True))
        sr_ref[...] = jnp.sum(xt, axis=0, keepdims=True) / jnp.maximum(rn, 1e-12)
        acc_ref[...] = jnp.zeros_like(acc_ref)

    @pl.when(j < NBA)
    def _phase_a():
        a = adj_col_ref[...]
        csq_acc = jnp.zeros((128, BCA), dtype=jnp.float32)
        for r in range(0, N, 128):
            c = a[r:r + 128, :]
            csq_acc = csq_acc + c * c
        csq = jnp.sum(csq_acc, axis=0, keepdims=True)
        s_blk = sr_ref[:, pl.ds(j * BCA, BCA)]
        t_row = s_blk / jnp.maximum(jnp.sqrt(csq), 1e-12)

        acc = acc_ref[...]
        for k in range(BCA // 128):
            acc = acc + a[:, k * 128:(k + 1) * 128] * t_row[:, k * 128:(k + 1) * 128]
        acc_ref[...] = acc

        @pl.when(j == NBA - 1)
        def _finalize():
            nm = jnp.sum(acc_ref[...], axis=1, keepdims=True)
            total = jnp.sum(nm)
            h = jax.lax.dot_general(
                embs_t_ref[...], w_ref[...], (((0,), (0,)), ((), ())),
                preferred_element_type=jnp.float32,
            )
            acc_ref[:, 0:D] = (nm * (1.0 / total)) * h

    @pl.when(j >= NBA)
    def _phase_b():
        o = jnp.dot(adj_row_ref[...], acc_ref[:, 0:D],
                    preferred_element_type=jnp.float32)
        out_ref[...] = jnp.where(o >= 0, o, NEG_SLOPE * o)


@jax.jit
def _run(adj_matrix, node_embs, W):
    embs_t = node_embs.T

    def _cap(v, hi):
        return jnp.minimum(v, hi)

    return pl.pallas_call(
        _fused,
        grid=(NBA + NBB,),
        in_specs=[
            pl.BlockSpec((N, BCA), lambda j: (0, _cap(j, NBA - 1))),
            pl.BlockSpec((BRB, N),
                         lambda j: (_cap(jnp.maximum(j - NBA, 0), NBB - 1), 0)),
            pl.BlockSpec((D, N), lambda j: (0, 0)),
            pl.BlockSpec((D, D), lambda j: (0, 0)),
        ],
        out_specs=pl.BlockSpec(
            (BRB, D), lambda j: (_cap(jnp.maximum(j - NBA, 0), NBB - 1), 0)),
        out_shape=jax.ShapeDtypeStruct((N, D), jnp.float32),
        scratch_shapes=[
            pltpu.VMEM((N, 128), jnp.float32),
            pltpu.VMEM((1, N), jnp.float32),
        ],
        compiler_params=pltpu.CompilerParams(
            dimension_semantics=("arbitrary",),
        ),
    )(adj_matrix, adj_matrix, embs_t, W)


def kernel(adj_matrix, node_embs, W):
    return _run(adj_matrix, node_embs, W)


# speedup vs baseline: 1.0362x; 1.0070x over previous
import jax
import jax.numpy as jnp
from jax.experimental import pallas as pl
from jax.experimental.pallas import tpu as pltpu

N = 8192
D = 64
BCA = 256
BRB = 256
NBA = N // BCA
NBB = N // BRB
NEG_SLOPE = 0.01


def _fused(adj_col_ref, adj_row_ref, embs_t_ref, w_ref, out_ref,
           acc_ref, sr_ref):
    j = pl.program_id(0)

    @pl.when(j == 0)
    def _init():
        xt = embs_t_ref[...]
        rn = jnp.sqrt(jnp.sum(xt * xt, axis=0, keepdims=True))
        sr_ref[...] = jnp.sum(xt, axis=0, keepdims=True) / jnp.maximum(rn, 1e-12)
        acc_ref[...] = jnp.zeros_like(acc_ref)

    @pl.when(j < NBA)
    def _phase_a():
        a = adj_col_ref[...]
        csq_acc = jnp.zeros((128, BCA), dtype=jnp.float32)
        for r in range(0, N, 128):
            c = a[r:r + 128, :]
            csq_acc = csq_acc + c * c
        csq = jnp.sum(csq_acc, axis=0, keepdims=True)
        s_blk = sr_ref[:, pl.ds(j * BCA, BCA)]
        t_row = s_blk / jnp.maximum(jnp.sqrt(csq), 1e-12)

        acc = acc_ref[...]
        for k in range(BCA // 128):
            acc = acc + a[:, k * 128:(k + 1) * 128] * t_row[:, k * 128:(k + 1) * 128]
        acc_ref[...] = acc

        @pl.when(j == NBA - 1)
        def _finalize():
            nm = jnp.sum(acc_ref[...], axis=1, keepdims=True)
            total = jnp.sum(nm)
            h = jax.lax.dot_general(
                embs_t_ref[...], w_ref[...], (((0,), (0,)), ((), ())),
                preferred_element_type=jnp.float32,
            )
            acc_ref[:, 0:D] = (nm * (1.0 / total)) * h

    @pl.when(j >= NBA - 1)
    def _phase_b():
        o = jnp.dot(adj_row_ref[...], acc_ref[:, 0:D],
                    preferred_element_type=jnp.float32)
        out_ref[...] = jnp.where(o >= 0, o, NEG_SLOPE * o)


@jax.jit
def _run(adj_matrix, node_embs, W):
    embs_t = node_embs.T

    def _cap(v, hi):
        return jnp.minimum(v, hi)

    return pl.pallas_call(
        _fused,
        grid=(NBA + NBB - 1,),
        in_specs=[
            pl.BlockSpec((N, BCA), lambda j: (0, _cap(j, NBA - 1))),
            pl.BlockSpec((BRB, N),
                         lambda j: (_cap(jnp.maximum(j - (NBA - 1), 0), NBB - 1), 0)),
            pl.BlockSpec((D, N), lambda j: (0, 0)),
            pl.BlockSpec((D, D), lambda j: (0, 0)),
        ],
        out_specs=pl.BlockSpec(
            (BRB, D), lambda j: (_cap(jnp.maximum(j - (NBA - 1), 0), NBB - 1), 0)),
        out_shape=jax.ShapeDtypeStruct((N, D), jnp.float32),
        scratch_shapes=[
            pltpu.VMEM((N, 128), jnp.float32),
            pltpu.VMEM((1, N), jnp.float32),
        ],
        compiler_params=pltpu.CompilerParams(
            dimension_semantics=("arbitrary",),
        ),
    )(adj_matrix, adj_matrix, embs_t, W)


def kernel(adj_matrix, node_embs, W):
    return _run(adj_matrix, node_embs, W)


# speedup vs baseline: 1.0393x; 1.0030x over previous
import jax
import jax.numpy as jnp
from jax.experimental import pallas as pl
from jax.experimental.pallas import tpu as pltpu

N = 8192
D = 64
BCA = 256
BRB = 256
NBA = N // BCA
NBB = N // BRB
NEG_SLOPE = 0.01


def _fused(adj_col_ref, adj_row_ref, embs_t_ref, w_ref, out_ref,
           acc_ref, sr_ref, h_ref):
    j = pl.program_id(0)

    @pl.when(j == 0)
    def _init():
        xt = embs_t_ref[...]
        rn = jnp.sqrt(jnp.sum(xt * xt, axis=0, keepdims=True))
        sr_ref[...] = jnp.sum(xt, axis=0, keepdims=True) / jnp.maximum(rn, 1e-12)
        acc_ref[...] = jnp.zeros_like(acc_ref)
        h_ref[...] = jax.lax.dot_general(
            xt, w_ref[...], (((0,), (0,)), ((), ())),
            preferred_element_type=jnp.float32,
        )

    @pl.when(j < NBA)
    def _phase_a():
        a = adj_col_ref[...]
        csq_acc = jnp.zeros((128, BCA), dtype=jnp.float32)
        for r in range(0, N, 128):
            c = a[r:r + 128, :]
            csq_acc = csq_acc + c * c
        csq = jnp.sum(csq_acc, axis=0, keepdims=True)
        s_blk = sr_ref[:, pl.ds(j * BCA, BCA)]
        t_row = s_blk / jnp.maximum(jnp.sqrt(csq), 1e-12)

        acc = acc_ref[...]
        for k in range(BCA // 128):
            acc = acc + a[:, k * 128:(k + 1) * 128] * t_row[:, k * 128:(k + 1) * 128]
        acc_ref[...] = acc

        @pl.when(j == NBA - 1)
        def _finalize():
            nm = jnp.sum(acc_ref[...], axis=1, keepdims=True)
            total = jnp.sum(nm)
            acc_ref[:, 0:D] = (nm * (1.0 / total)) * h_ref[...]

    @pl.when(j >= NBA - 1)
    def _phase_b():
        o = jnp.dot(adj_row_ref[...], acc_ref[:, 0:D],
                    preferred_element_type=jnp.float32)
        out_ref[...] = jnp.where(o >= 0, o, NEG_SLOPE * o)


@jax.jit
def _run(adj_matrix, node_embs, W):
    embs_t = node_embs.T

    def _cap(v, hi):
        return jnp.minimum(v, hi)

    return pl.pallas_call(
        _fused,
        grid=(NBA + NBB - 1,),
        in_specs=[
            pl.BlockSpec((N, BCA), lambda j: (0, _cap(j, NBA - 1))),
            pl.BlockSpec((BRB, N),
                         lambda j: (_cap(jnp.maximum(j - (NBA - 1), 0), NBB - 1), 0)),
            pl.BlockSpec((D, N), lambda j: (0, 0)),
            pl.BlockSpec((D, D), lambda j: (0, 0)),
        ],
        out_specs=pl.BlockSpec(
            (BRB, D), lambda j: (_cap(jnp.maximum(j - (NBA - 1), 0), NBB - 1), 0)),
        out_shape=jax.ShapeDtypeStruct((N, D), jnp.float32),
        scratch_shapes=[
            pltpu.VMEM((N, 128), jnp.float32),
            pltpu.VMEM((1, N), jnp.float32),
            pltpu.VMEM((N, D), jnp.float32),
        ],
        compiler_params=pltpu.CompilerParams(
            dimension_semantics=("arbitrary",),
        ),
    )(adj_matrix, adj_matrix, embs_t, W)


def kernel(adj_matrix, node_embs, W):
    return _run(adj_matrix, node_embs, W)
